# baseline (speedup 1.0000x reference)
import functools
import math

import jax
import jax.numpy as jnp
from jax import lax
from jax.experimental import pallas as pl
from jax.experimental.pallas import tpu as pltpu

EPS = 1e-7

_ATAN_COEFS = (
    9.9999999375e-01, -3.3333137975e-01, 1.9993694319e-01, -1.4211106055e-01,
    1.0667486906e-01, -7.5569002114e-02, 4.3278241863e-02, -1.6413190479e-02,
    2.9327619590e-03,
)


def _atan_pos(x):
    y = jnp.minimum(x, 1.0)
    r = 1.0 / jnp.maximum(x, 1.0)
    y2 = y * y
    r2 = r * r
    py = _ATAN_COEFS[-1]
    pr = _ATAN_COEFS[-1]
    for c in _ATAN_COEFS[-2::-1]:
        py = py * y2 + c
        pr = pr * r2 + c
    small = y * py
    big = (math.pi / 2) - r * pr
    return jnp.where(x <= 1.0, small, big)


def _bce_body(p_ref, t_ref, out_ref):
    i = pl.program_id(0)
    p = p_ref[...]
    t = t_ref[...]
    bce = jnp.maximum(p, 0.0) - p * t + jnp.log1p(jnp.exp(-jnp.abs(p)))
    partial = jnp.sum(bce)

    @pl.when(i == 0)
    def _():
        out_ref[0, 0] = partial

    @pl.when(i > 0)
    def _():
        out_ref[0, 0] += partial


def _ciou_loss(px1, py1, px2, py2, tx1, ty1, tx2, ty2, w):
    xmin_i = jnp.maximum(px1, tx1)
    ymin_i = jnp.maximum(py1, ty1)
    xmax_i = jnp.minimum(px2, tx2)
    ymax_i = jnp.minimum(py2, ty2)
    inter = jnp.clip(xmax_i - xmin_i, 0) * jnp.clip(ymax_i - ymin_i, 0)
    a1 = (px2 - px1) * (py2 - py1)
    a2 = (tx2 - tx1) * (ty2 - ty1)
    union = a1 + a2 - inter
    iou = inter / (union + EPS)
    cent = ((px2 + px1) - (tx2 + tx1)) ** 2 + ((py2 + py1) - (ty2 + ty1)) ** 2
    c_x = jnp.maximum(px2, tx2) - jnp.minimum(px1, tx1)
    c_y = jnp.maximum(py2, ty2) - jnp.minimum(py1, ty1)
    diag = 4.0 * (c_x**2 + c_y**2) + EPS * 4.0
    diou = iou - cent / diag
    arct = _atan_pos((px2 - px1) / (py2 - py1 + EPS)) - _atan_pos(
        (tx2 - tx1) / (ty2 - ty1 + EPS))
    v = (4.0 / math.pi**2) * arct * arct
    alpha = v / (v - iou + 1.0 + EPS)
    ciou = diou - alpha * v
    return (1.0 - ciou) * w


def _box_body(c_ref, out_ref):
    comp = [c_ref[k] for k in range(9)]
    out_ref[0, 0] = jnp.sum(_ciou_loss(*comp))


def _bce_sum(p2, t2, block_rows):
    rows = p2.shape[0]
    grid = rows // block_rows
    return pl.pallas_call(
        _bce_body,
        grid=(grid,),
        in_specs=[
            pl.BlockSpec((block_rows, 128), lambda i: (i, 0)),
            pl.BlockSpec((block_rows, 128), lambda i: (i, 0)),
        ],
        out_specs=pl.BlockSpec(memory_space=pltpu.SMEM),
        out_shape=jax.ShapeDtypeStruct((1, 1), jnp.float32),
    )(p2, t2)


def _box_sum(comps):
    return pl.pallas_call(
        _box_body,
        out_specs=pl.BlockSpec(memory_space=pltpu.SMEM),
        out_shape=jax.ShapeDtypeStruct((1, 1), jnp.float32),
    )(comps)


def kernel(predicts_cls, predicts_bbox, targets_cls, targets_bbox,
           valid_masks, box_norm, cls_norm):
    B, A, C = predicts_cls.shape
    n_cls = B * A * C
    p2 = predicts_cls.reshape(n_cls // 128, 128)
    t2 = targets_cls.reshape(n_cls // 128, 128)
    bce_sum = _bce_sum(p2, t2, block_rows=4200)

    n_box = B * A
    pad = (-n_box) % (128 * 8)
    npad = n_box + pad

    pb = predicts_bbox.reshape(n_box, 4)
    tb = targets_bbox.reshape(n_box, 4)
    w = valid_masks.reshape(n_box).astype(jnp.float32) * box_norm.reshape(n_box)
    comps = jnp.stack([
        pb[:, 0], pb[:, 1], pb[:, 2], pb[:, 3],
        tb[:, 0], tb[:, 1], tb[:, 2], tb[:, 3], w,
    ])
    comps = jnp.pad(comps, ((0, 0), (0, pad))).reshape(9, npad // 128, 128)
    iou_sum = _box_sum(comps)

    loss_cls = bce_sum[0, 0] / cls_norm
    loss_iou = iou_sum[0, 0] / cls_norm
    return (loss_cls, loss_iou)


# speedup vs baseline: 2.0867x; 2.0867x over previous
import functools
import math

import jax
import jax.numpy as jnp
from jax import lax
from jax.experimental import pallas as pl
from jax.experimental.pallas import tpu as pltpu

EPS = 1e-7

_ATAN_COEFS = (
    9.9999999375e-01, -3.3333137975e-01, 1.9993694319e-01, -1.4211106055e-01,
    1.0667486906e-01, -7.5569002114e-02, 4.3278241863e-02, -1.6413190479e-02,
    2.9327619590e-03,
)


def _atan_pos(x):
    y = jnp.minimum(x, 1.0)
    r = 1.0 / jnp.maximum(x, 1.0)
    y2 = y * y
    r2 = r * r
    py = _ATAN_COEFS[-1]
    pr = _ATAN_COEFS[-1]
    for c in _ATAN_COEFS[-2::-1]:
        py = py * y2 + c
        pr = pr * r2 + c
    small = y * py
    big = (math.pi / 2) - r * pr
    return jnp.where(x <= 1.0, small, big)


def _ciou_loss(px1, py1, px2, py2, tx1, ty1, tx2, ty2, w):
    xmin_i = jnp.maximum(px1, tx1)
    ymin_i = jnp.maximum(py1, ty1)
    xmax_i = jnp.minimum(px2, tx2)
    ymax_i = jnp.minimum(py2, ty2)
    inter = jnp.clip(xmax_i - xmin_i, 0) * jnp.clip(ymax_i - ymin_i, 0)
    a1 = (px2 - px1) * (py2 - py1)
    a2 = (tx2 - tx1) * (ty2 - ty1)
    union = a1 + a2 - inter
    iou = inter / (union + EPS)
    cent = ((px2 + px1) - (tx2 + tx1)) ** 2 + ((py2 + py1) - (ty2 + ty1)) ** 2
    c_x = jnp.maximum(px2, tx2) - jnp.minimum(px1, tx1)
    c_y = jnp.maximum(py2, ty2) - jnp.minimum(py1, ty1)
    diag = 4.0 * (c_x**2 + c_y**2) + 4.0 * EPS
    diou = iou - cent / diag
    arct = _atan_pos((px2 - px1) / (py2 - py1 + EPS)) - _atan_pos(
        (tx2 - tx1) / (ty2 - ty1 + EPS))
    v = (4.0 / math.pi**2) * arct * arct
    alpha = v / (v - iou + 1.0 + EPS)
    ciou = diou - alpha * v
    return (1.0 - ciou) * w


def _body(p_ref, t_ref, px1, py1, px2, py2, tx1, ty1, tx2, ty2, m_ref,
          bn_ref, cls_ref, iou_ref, *, grid_j):
    i = pl.program_id(0)
    j = pl.program_id(1)
    p = p_ref[0]
    t = t_ref[0]
    bce = jnp.maximum(p, 0.0) - p * t + jnp.log1p(jnp.exp(-jnp.abs(p)))
    partial = jnp.sum(bce)

    @pl.when(jnp.logical_and(i == 0, j == 0))
    def _():
        cls_ref[0, 0] = partial
        iou_ref[0, 0] = 0.0

    @pl.when(jnp.logical_or(i > 0, j > 0))
    def _():
        cls_ref[0, 0] += partial

    @pl.when(jnp.logical_and(i == pl.num_programs(0) - 1, j == grid_j - 1))
    def _():
        w = m_ref[...] * bn_ref[...]
        el = _ciou_loss(px1[...], py1[...], px2[...], py2[...],
                        tx1[...], ty1[...], tx2[...], ty2[...], w)
        iou_ref[0, 0] += jnp.sum(el)


def kernel(predicts_cls, predicts_bbox, targets_cls, targets_bbox,
           valid_masks, box_norm, cls_norm):
    B, A, C = predicts_cls.shape
    grid_j = 6
    blk = A // grid_j

    comps = ([predicts_bbox[:, :, k] for k in range(4)]
             + [targets_bbox[:, :, k] for k in range(4)])
    m = valid_masks.astype(jnp.float32)

    cls_spec = pl.BlockSpec((1, blk, C), lambda i, j: (i, j, 0))
    full_spec = pl.BlockSpec((B, A), lambda i, j: (0, 0))
    out_spec = pl.BlockSpec(memory_space=pltpu.SMEM)

    cls_sum, iou_sum = pl.pallas_call(
        functools.partial(_body, grid_j=grid_j),
        grid=(B, grid_j),
        in_specs=[cls_spec, cls_spec] + [full_spec] * 10,
        out_specs=[out_spec, out_spec],
        out_shape=[jax.ShapeDtypeStruct((1, 1), jnp.float32)] * 2,
    )(predicts_cls, targets_cls, *comps, m, box_norm)

    loss_cls = cls_sum[0, 0] / cls_norm
    loss_iou = iou_sum[0, 0] / cls_norm
    return (loss_cls, loss_iou)
